# Initial kernel scaffold; baseline (speedup 1.0000x reference)
import jax
import jax.numpy as jnp
from jax.experimental import pallas as pl


def kernel(inputs, L):
    raise NotImplementedError("write your pallas kernel here")


# speedup vs baseline: 1.5460x; 1.5460x over previous
import functools

import jax
import jax.numpy as jnp
from jax import lax
from jax.experimental import pallas as pl
from jax.experimental.pallas import tpu as pltpu
from jax.experimental.pallas import tpu_sc as plsc

VOCAB = 1_000_000
DIM = 32
ROWS_TOTAL = 16384 * 26

_INFO = plsc.get_sparse_core_info()
NC = _INFO.num_cores
NS = _INFO.num_subcores
NW = NC * NS
PER_W = ROWS_TOTAL // NW
CHUNK = 1024
NCHUNK = PER_W // CHUNK


@functools.partial(
    pl.kernel,
    out_type=jax.ShapeDtypeStruct((ROWS_TOTAL, DIM), jnp.float32),
    mesh=plsc.VectorSubcoreMesh(core_axis_name="c", subcore_axis_name="s"),
    compiler_params=pltpu.CompilerParams(use_tc_tiling_on_sc=False),
    scratch_types=[
        pltpu.VMEM((CHUNK,), jnp.int32),
        pltpu.VMEM((CHUNK, DIM), jnp.float32),
        pltpu.SemaphoreType.DMA,
    ],
)
def _gather_kernel(idx_hbm, table_hbm, out_hbm, idx_v, rows_v, sem):
    wid = lax.axis_index("s") * NC + lax.axis_index("c")
    base = wid * PER_W
    for g in range(NCHUNK):
        off = base + g * CHUNK
        pltpu.sync_copy(idx_hbm.at[pl.ds(off, CHUNK)], idx_v)
        pltpu.async_copy(table_hbm.at[idx_v], rows_v, sem).wait()
        pltpu.sync_copy(rows_v, out_hbm.at[pl.ds(off, CHUNK)])


def kernel(inputs, L):
    flat_idx = inputs.reshape(-1).astype(jnp.int32)
    out = _gather_kernel(flat_idx, L)
    return out.reshape(inputs.shape[0], inputs.shape[1], DIM)


# speedup vs baseline: 1.5786x; 1.0211x over previous
import functools

import jax
import jax.numpy as jnp
from jax import lax
from jax.experimental import pallas as pl
from jax.experimental.pallas import tpu as pltpu
from jax.experimental.pallas import tpu_sc as plsc

VOCAB = 1_000_000
DIM = 32
ROWS_TOTAL = 16384 * 26

_INFO = plsc.get_sparse_core_info()
NC = _INFO.num_cores
NS = _INFO.num_subcores
NW = NC * NS
PER_W = ROWS_TOTAL // NW
CHUNK = 1664
NCHUNK = PER_W // CHUNK


@functools.partial(
    pl.kernel,
    out_type=jax.ShapeDtypeStruct((ROWS_TOTAL, DIM), jnp.float32),
    mesh=plsc.VectorSubcoreMesh(core_axis_name="c", subcore_axis_name="s"),
    compiler_params=pltpu.CompilerParams(use_tc_tiling_on_sc=False),
    scratch_types=[
        pltpu.VMEM((PER_W,), jnp.int32),
        pltpu.VMEM((2, CHUNK, DIM), jnp.float32),
        pltpu.SemaphoreType.DMA,
        pltpu.SemaphoreType.DMA,
    ],
)
def _gather_kernel(idx_hbm, table_hbm, out_hbm, idx_v, rows_v, gsem, osem):
    wid = lax.axis_index("s") * NC + lax.axis_index("c")
    base = wid * PER_W
    pltpu.sync_copy(idx_hbm.at[pl.ds(base, PER_W)], idx_v)

    def gather(g):
        return pltpu.async_copy(
            table_hbm.at[idx_v.at[pl.ds(g * CHUNK, CHUNK)]],
            rows_v.at[g % 2],
            gsem,
        )

    def writeout(g):
        return pltpu.async_copy(
            rows_v.at[g % 2],
            out_hbm.at[pl.ds(base + g * CHUNK, CHUNK)],
            osem,
        )

    gathers = [gather(0)]
    outs = []
    for g in range(1, NCHUNK):
        if g >= 2:
            outs[g - 2].wait()
        gathers.append(gather(g))
        gathers[g - 1].wait()
        outs.append(writeout(g - 1))
    gathers[NCHUNK - 1].wait()
    outs.append(writeout(NCHUNK - 1))
    outs[NCHUNK - 2].wait()
    outs[NCHUNK - 1].wait()


def kernel(inputs, L):
    flat_idx = inputs.reshape(-1).astype(jnp.int32)
    out = _gather_kernel(flat_idx, L)
    return out.reshape(inputs.shape[0], inputs.shape[1], DIM)


# speedup vs baseline: 2.0079x; 1.2720x over previous
import functools

import jax
import jax.numpy as jnp
from jax import lax
from jax.experimental import pallas as pl
from jax.experimental.pallas import tpu as pltpu
from jax.experimental.pallas import tpu_sc as plsc

VOCAB = 1_000_000
DIM = 32
B = 16384
F = 26
BB = B // 128
ITEMS = F * BB
ROWS_TOTAL = B * F

_INFO = plsc.get_sparse_core_info()
NC = _INFO.num_cores
NS = _INFO.num_subcores
NW = NC * NS
PER_W = ITEMS // NW
PITCH = 129


@functools.partial(
    pl.kernel,
    out_type=jax.ShapeDtypeStruct((F, 4, BB, 8, 128), jnp.float32),
    mesh=plsc.VectorSubcoreMesh(core_axis_name="c", subcore_axis_name="s"),
    compiler_params=pltpu.CompilerParams(
        use_tc_tiling_on_sc=False, needs_layout_passes=False
    ),
    scratch_types=[
        pltpu.VMEM((PER_W * 128,), jnp.int32),
        pltpu.VMEM((2, 128, DIM), jnp.float32),
        pltpu.VMEM((2, 4, 8, PITCH), jnp.float32),
        pltpu.SemaphoreType.DMA,
        pltpu.SemaphoreType.DMA,
    ],
)
def _gather_kernel(idx_hbm, table2d, out_hbm, idx_v, rows_v, trows_v, gsem, osem):
    wid = lax.axis_index("s") * NC + lax.axis_index("c")
    t0 = wid * PER_W
    pltpu.sync_copy(idx_hbm.at[pl.ds(t0 * 128, PER_W * 128)], idx_v)

    iota = lax.iota(jnp.int32, 16)
    ds_idx = lax.rem(iota, 8)
    db_lo = lax.div(iota, 8)
    db_hi = db_lo + 2

    def gather(t, buf):
        return pltpu.async_copy(
            table2d.at[idx_v.at[pl.ds(t * 128, 128)]],
            rows_v.at[buf],
            gsem,
        )

    def transpose(buf):
        for b in range(128):
            b_vec = jnp.full((16,), b, jnp.int32)
            for h in range(2):
                v = rows_v[buf, b, pl.ds(16 * h, 16)]
                plsc.store_scatter(
                    trows_v.at[buf],
                    [db_hi if h else db_lo, ds_idx, b_vec],
                    v,
                )

    def writeout(t, buf):
        g = t0 + t
        f = g // BB
        bb = g - f * BB
        src = trows_v.at[buf, :, :, pl.ds(0, 128)]
        return pltpu.async_copy(src, out_hbm.at[f, :, bb], osem)

    def drain_out(t, buf):
        g = t0 + t
        f = g // BB
        bb = g - f * BB
        src = trows_v.at[buf, :, :, pl.ds(0, 128)]
        pltpu.make_async_copy(src, out_hbm.at[f, :, bb], osem).wait()

    def drain_gather(t, buf):
        pltpu.make_async_copy(
            table2d.at[idx_v.at[pl.ds(t * 128, 128)]],
            rows_v.at[buf],
            gsem,
        ).wait()

    gather(0, 0)

    def body(t, carry):
        for buf in range(2):
            tt = t + buf
            drain_gather(tt, buf)
            nxt = tt + 1

            @pl.when(nxt < PER_W)
            def _():
                gather(nxt, 1 - buf)

            @pl.when(tt >= 2)
            def _():
                drain_out(tt - 2, buf)

            transpose(buf)
            writeout(tt, buf)
        return carry

    lax.fori_loop(0, PER_W // 2, lambda i, c: body(2 * i, c), 0, unroll=False)
    drain_out(PER_W - 2, 0)
    drain_out(PER_W - 1, 1)


def kernel(inputs, L):
    idx_t = inputs.T.reshape(-1).astype(jnp.int32)
    p5 = _gather_kernel(idx_t, L)
    return p5.transpose(2, 4, 0, 1, 3).reshape(B, F, DIM)
